# baseline (speedup 1.0000x reference)
import functools

import jax
import jax.numpy as jnp
from jax import lax
from jax.experimental import pallas as pl
from jax.experimental.pallas import tpu as pltpu
from jax.experimental.pallas import tpu_sc as plsc

B = 512
M = 100000
D = 128
C = 1000
TEMP = 0.05

NC = 2
NS = 16
NW = NC * NS

CP = 1024
RPS = CP // NS
CH = 128
NFULL = M // CH
TAIL = M - NFULL * CH
TAIL_OFF = NFULL * CH
ITERS = (NFULL + NW - 1) // NW
CW = 128
TPW = B // NW


def _sc_body(features_hbm, labels_hbm, indexes_hbm,
             zsum_hbm, ones_hbm,
             sums_out, cnts_out, targets_out,
             rows0_v, rows1_v, idx0_v, idx1_v, idxt_v, ones_v, tidx_v,
             tgt_v, acc_sh, cnt_sh, sem, ls0, ls1, ss0, ss1):
    cid = lax.axis_index("c")
    sid = lax.axis_index("s")
    w = cid * NS + sid

    tb = w * TPW
    pltpu.sync_copy(indexes_hbm.at[pl.ds(tb, TPW)], tidx_v)
    pltpu.async_copy(labels_hbm.at[tidx_v], tgt_v, sem).wait()
    pltpu.sync_copy(tgt_v, targets_out.at[pl.ds(tb, TPW)])

    pltpu.sync_copy(zsum_hbm.at[pl.ds(sid * RPS, RPS)],
                    acc_sh.at[pl.ds(sid * RPS, RPS)])
    pltpu.sync_copy(zsum_hbm.at[pl.ds(sid * RPS, RPS)],
                    cnt_sh.at[pl.ds(sid * RPS, RPS)])
    pltpu.sync_copy(ones_hbm, ones_v)
    plsc.subcore_barrier()

    def chunk_pair(i2, carry):
        j0 = w + (2 * i2) * NW
        j1 = j0 + NW

        @pl.when(j1 < NFULL)
        def _():
            cr0 = pltpu.async_copy(
                features_hbm.at[pl.ds(j0 * CH, CH)], rows0_v, ls0)
            ci0 = pltpu.async_copy(
                labels_hbm.at[pl.ds(j0 * CH, CH)], idx0_v, ls0)
            cr1 = pltpu.async_copy(
                features_hbm.at[pl.ds(j1 * CH, CH)], rows1_v, ls1)
            ci1 = pltpu.async_copy(
                labels_hbm.at[pl.ds(j1 * CH, CH)], idx1_v, ls1)
            cr0.wait()
            ci0.wait()
            s0 = pltpu.async_copy(rows0_v, acc_sh.at[idx0_v], ss0, add=True)
            c0 = pltpu.async_copy(ones_v, cnt_sh.at[idx0_v], ss0, add=True)
            cr1.wait()
            ci1.wait()
            s1 = pltpu.async_copy(rows1_v, acc_sh.at[idx1_v], ss1, add=True)
            c1 = pltpu.async_copy(ones_v, cnt_sh.at[idx1_v], ss1, add=True)
            s0.wait()
            c0.wait()
            s1.wait()
            c1.wait()

        @pl.when(jnp.logical_and(j0 < NFULL, j1 >= NFULL))
        def _():
            off = j0 * CH
            pltpu.sync_copy(features_hbm.at[pl.ds(off, CH)], rows0_v)
            pltpu.sync_copy(labels_hbm.at[pl.ds(off, CH)], idx0_v)
            pltpu.sync_copy(rows0_v, acc_sh.at[idx0_v], add=True)
            pltpu.sync_copy(ones_v, cnt_sh.at[idx0_v], add=True)

        return carry

    lax.fori_loop(0, (ITERS + 1) // 2, chunk_pair, 0)

    @pl.when(w == NW - 1)
    def _():
        pltpu.sync_copy(features_hbm.at[pl.ds(TAIL_OFF, TAIL)],
                        rows0_v.at[pl.ds(0, TAIL)])
        pltpu.sync_copy(labels_hbm.at[pl.ds(TAIL_OFF, TAIL)], idxt_v)
        pltpu.sync_copy(rows0_v.at[pl.ds(0, TAIL)], acc_sh.at[idxt_v],
                        add=True)
        pltpu.sync_copy(ones_v.at[pl.ds(0, TAIL)], cnt_sh.at[idxt_v],
                        add=True)

    plsc.subcore_barrier()

    base = cid * CP + sid * RPS
    pltpu.sync_copy(acc_sh.at[pl.ds(sid * RPS, RPS)],
                    sums_out.at[pl.ds(base, RPS)])
    pltpu.sync_copy(cnt_sh.at[pl.ds(sid * RPS, RPS)],
                    cnts_out.at[pl.ds(base, RPS)])


@functools.cache
def _get_sc_call():
  return pl.kernel(
    _sc_body,
    out_type=[
        jax.ShapeDtypeStruct((NC * CP, D), jnp.float32),
        jax.ShapeDtypeStruct((NC * CP, CW), jnp.float32),
        jax.ShapeDtypeStruct((B,), jnp.int32),
    ],
    mesh=plsc.VectorSubcoreMesh(
        core_axis_name="c", subcore_axis_name="s",
        num_cores=NC, num_subcores=NS),
    scratch_types=[
        pltpu.VMEM((CH, D), jnp.float32),
        pltpu.VMEM((CH, D), jnp.float32),
        pltpu.VMEM((CH,), jnp.int32),
        pltpu.VMEM((CH,), jnp.int32),
        pltpu.VMEM((TAIL,), jnp.int32),
        pltpu.VMEM((CH, CW), jnp.float32),
        pltpu.VMEM((TPW,), jnp.int32),
        pltpu.VMEM((TPW,), jnp.int32),
        pltpu.VMEM_SHARED((CP, D), jnp.float32),
        pltpu.VMEM_SHARED((CP, CW), jnp.float32),
        pltpu.SemaphoreType.DMA,
        pltpu.SemaphoreType.DMA,
        pltpu.SemaphoreType.DMA,
        pltpu.SemaphoreType.DMA,
        pltpu.SemaphoreType.DMA,
    ],
  )


def _tc_body(feat_ref, sums_ref, cnts_ref, tgt_ref, out_ref):
    feat = feat_ref[...]
    nrm = jnp.sqrt(jnp.sum(feat * feat, axis=1, keepdims=True))
    x = feat / jnp.maximum(nrm, 1e-12)

    cs = sums_ref[0:CP, :] + sums_ref[CP:2 * CP, :]
    cnt = cnts_ref[0:CP, :] + cnts_ref[CP:2 * CP, :]

    sim = lax.dot_general(
        x, cs, (((1,), (1,)), ((), ())),
        preferred_element_type=jnp.float32,
        precision=lax.Precision.HIGHEST) * (1.0 / TEMP)

    onesk = jnp.full((1, CW), 1.0 / CW, jnp.float32)
    cnt_row = lax.dot_general(
        onesk, cnt, (((1,), (1,)), ((), ())),
        preferred_element_type=jnp.float32,
        precision=lax.Precision.HIGHEST)

    mask = (cnt_row > 0.0).astype(jnp.float32)
    denom = mask * cnt_row + (1.0 - mask)
    simd = sim / denom
    exps = jnp.exp(simd) * mask
    ssum = jnp.sum(exps, axis=1, keepdims=True) + 1e-06
    p = exps / ssum
    logp = jnp.log(p + 1e-06)

    tgt = tgt_ref[...]
    oh = (lax.broadcasted_iota(jnp.int32, (B, CP), 1) == tgt)
    loss = -jnp.sum(jnp.where(oh, logp, 0.0)) * (1.0 / B)
    out_ref[0, 0] = loss


_tc_call = pl.pallas_call(
    _tc_body,
    out_shape=jax.ShapeDtypeStruct((1, 1), jnp.float32),
    out_specs=pl.BlockSpec(memory_space=pltpu.SMEM),
)


def kernel(feat, indexes, features, labels):
    zsum = jnp.zeros((CP, D), jnp.float32)
    ones = jnp.ones((CH, CW), jnp.float32)
    sums, cnts, targets = _get_sc_call()(
        features, labels, indexes, zsum, ones)
    loss = _tc_call(feat, sums, cnts, targets.reshape(B, 1))
    return loss.reshape(())


# speedup vs baseline: 1.2966x; 1.2966x over previous
import functools

import jax
import jax.numpy as jnp
from jax import lax
from jax.experimental import pallas as pl
from jax.experimental.pallas import tpu as pltpu
from jax.experimental.pallas import tpu_sc as plsc

B = 512
M = 100000
D = 128
C = 1000
TEMP = 0.05

NC = 2
NS = 16
NW = NC * NS

CP = 1024
RPS = CP // NS
CH = 128
NFULL = M // CH
TAIL = M - NFULL * CH
TAIL_OFF = NFULL * CH
ITERS = (NFULL + NW - 1) // NW
CW = 128
TPW = B // NW


def _sc_body(features_hbm, labels_hbm, indexes_hbm,
             zsum_hbm,
             sums_out, cnts_out, targets_out,
             rows0_v, rows1_v, idx0_v, idx1_v, idxt_v, tidx_v,
             tgt_v, hist_v, acc_sh, sem, ls0, ls1, ss0, ss1):
    cid = lax.axis_index("c")
    sid = lax.axis_index("s")
    w = cid * NS + sid

    tb = w * TPW
    pltpu.sync_copy(indexes_hbm.at[pl.ds(tb, TPW)], tidx_v)
    pltpu.async_copy(labels_hbm.at[tidx_v], tgt_v, sem).wait()
    pltpu.sync_copy(tgt_v, targets_out.at[pl.ds(tb, TPW)])

    pltpu.sync_copy(zsum_hbm.at[pl.ds(sid * RPS, RPS)],
                    acc_sh.at[pl.ds(sid * RPS, RPS)])
    z16 = jnp.zeros((16,), jnp.float32)
    for r in range(CP // 16):
        hist_v[pl.ds(r * 16, 16)] = z16
    plsc.subcore_barrier()

    ones16 = jnp.ones((16,), jnp.float32)

    def bump(idx_ref, n):
        for r in range(n // 16):
            iv = idx_ref[pl.ds(r * 16, 16)]
            plsc.addupdate_scatter(hist_v, [iv], ones16)

    def chunk_pair(i2, carry):
        j0 = w + (2 * i2) * NW
        j1 = j0 + NW

        @pl.when(j1 < NFULL)
        def _():
            cr0 = pltpu.async_copy(
                features_hbm.at[pl.ds(j0 * CH, CH)], rows0_v, ls0)
            ci0 = pltpu.async_copy(
                labels_hbm.at[pl.ds(j0 * CH, CH)], idx0_v, ls0)
            cr1 = pltpu.async_copy(
                features_hbm.at[pl.ds(j1 * CH, CH)], rows1_v, ls1)
            ci1 = pltpu.async_copy(
                labels_hbm.at[pl.ds(j1 * CH, CH)], idx1_v, ls1)
            cr0.wait()
            ci0.wait()
            s0 = pltpu.async_copy(rows0_v, acc_sh.at[idx0_v], ss0, add=True)
            bump(idx0_v, CH)
            cr1.wait()
            ci1.wait()
            s1 = pltpu.async_copy(rows1_v, acc_sh.at[idx1_v], ss1, add=True)
            bump(idx1_v, CH)
            s0.wait()
            s1.wait()

        @pl.when(jnp.logical_and(j0 < NFULL, j1 >= NFULL))
        def _():
            off = j0 * CH
            pltpu.sync_copy(features_hbm.at[pl.ds(off, CH)], rows0_v)
            pltpu.sync_copy(labels_hbm.at[pl.ds(off, CH)], idx0_v)
            pltpu.sync_copy(rows0_v, acc_sh.at[idx0_v], add=True)
            bump(idx0_v, CH)

        return carry

    lax.fori_loop(0, (ITERS + 1) // 2, chunk_pair, 0)

    @pl.when(w == NW - 1)
    def _():
        pltpu.sync_copy(features_hbm.at[pl.ds(TAIL_OFF, TAIL)],
                        rows0_v.at[pl.ds(0, TAIL)])
        pltpu.sync_copy(labels_hbm.at[pl.ds(TAIL_OFF, TAIL)], idxt_v)
        pltpu.sync_copy(rows0_v.at[pl.ds(0, TAIL)], acc_sh.at[idxt_v],
                        add=True)
        bump(idxt_v, TAIL)

    plsc.subcore_barrier()

    base = cid * CP + sid * RPS
    pltpu.sync_copy(acc_sh.at[pl.ds(sid * RPS, RPS)],
                    sums_out.at[pl.ds(base, RPS)])
    pltpu.sync_copy(hist_v, cnts_out.at[w])


@functools.cache
def _get_sc_call():
  return pl.kernel(
    _sc_body,
    out_type=[
        jax.ShapeDtypeStruct((NC * CP, D), jnp.float32),
        jax.ShapeDtypeStruct((NW, CP), jnp.float32),
        jax.ShapeDtypeStruct((B,), jnp.int32),
    ],
    mesh=plsc.VectorSubcoreMesh(
        core_axis_name="c", subcore_axis_name="s",
        num_cores=NC, num_subcores=NS),
    scratch_types=[
        pltpu.VMEM((CH, D), jnp.float32),
        pltpu.VMEM((CH, D), jnp.float32),
        pltpu.VMEM((CH,), jnp.int32),
        pltpu.VMEM((CH,), jnp.int32),
        pltpu.VMEM((TAIL,), jnp.int32),
        pltpu.VMEM((TPW,), jnp.int32),
        pltpu.VMEM((TPW,), jnp.int32),
        pltpu.VMEM((CP,), jnp.float32),
        pltpu.VMEM_SHARED((CP, D), jnp.float32),
        pltpu.SemaphoreType.DMA,
        pltpu.SemaphoreType.DMA,
        pltpu.SemaphoreType.DMA,
        pltpu.SemaphoreType.DMA,
        pltpu.SemaphoreType.DMA,
    ],
    compiler_params=pltpu.CompilerParams(needs_layout_passes=False),
  )


def _tc_body(feat_ref, sums_ref, cnts_ref, tgt_ref, out_ref):
    feat = feat_ref[...]
    nrm = jnp.sqrt(jnp.sum(feat * feat, axis=1, keepdims=True))
    x = feat / jnp.maximum(nrm, 1e-12)

    cs = sums_ref[0:CP, :] + sums_ref[CP:2 * CP, :]

    sim = lax.dot_general(
        x, cs, (((1,), (1,)), ((), ())),
        preferred_element_type=jnp.float32,
        precision=lax.Precision.HIGHEST) * (1.0 / TEMP)

    onesk = jnp.ones((1, NW), jnp.float32)
    cnt_row = lax.dot_general(
        onesk, cnts_ref[...], (((1,), (0,)), ((), ())),
        preferred_element_type=jnp.float32,
        precision=lax.Precision.HIGHEST)

    mask = (cnt_row > 0.0).astype(jnp.float32)
    denom = mask * cnt_row + (1.0 - mask)
    simd = sim / denom
    exps = jnp.exp(simd) * mask
    ssum = jnp.sum(exps, axis=1, keepdims=True) + 1e-06
    p = exps / ssum
    logp = jnp.log(p + 1e-06)

    tgt = tgt_ref[...]
    oh = (lax.broadcasted_iota(jnp.int32, (B, CP), 1) == tgt)
    loss = -jnp.sum(jnp.where(oh, logp, 0.0)) * (1.0 / B)
    out_ref[0, 0] = loss


_tc_call = pl.pallas_call(
    _tc_body,
    out_shape=jax.ShapeDtypeStruct((1, 1), jnp.float32),
    out_specs=pl.BlockSpec(memory_space=pltpu.SMEM),
)


def kernel(feat, indexes, features, labels):
    zsum = jnp.zeros((CP, D), jnp.float32)
    sums, cnts, targets = _get_sc_call()(
        features, labels, indexes, zsum)
    loss = _tc_call(feat, sums, cnts, targets.reshape(B, 1))
    return loss.reshape(())


# speedup vs baseline: 1.5656x; 1.2074x over previous
import functools

import jax
import jax.numpy as jnp
from jax import lax
from jax.experimental import pallas as pl
from jax.experimental.pallas import tpu as pltpu
from jax.experimental.pallas import tpu_sc as plsc

B = 512
M = 100000
D = 128
C = 1000
TEMP = 0.05

NC = 2
NS = 16
NW = NC * NS

CP = 1024
RPS = CP // NS
CH = 128
NFULL = M // CH
TAIL = M - NFULL * CH
TAIL_OFF = NFULL * CH
ITERS = (NFULL + NW - 1) // NW
CW = 128
TPW = B // NW


def _sc_body(features_hbm, labels_hbm, indexes_hbm,
             zsum_hbm,
             sums_out, cnts_out, targets_out,
             rows0_v, rows1_v, rows2_v, rows3_v,
             idx0_v, idx1_v, idx2_v, idx3_v, idxt_v, tidx_v,
             tgt_v, hist_v, acc_sh, sem,
             ls0, ls1, ls2, ls3, ss0, ss1, ss2, ss3):
    cid = lax.axis_index("c")
    sid = lax.axis_index("s")
    w = cid * NS + sid

    tb = w * TPW
    pltpu.sync_copy(indexes_hbm.at[pl.ds(tb, TPW)], tidx_v)
    pltpu.async_copy(labels_hbm.at[tidx_v], tgt_v, sem).wait()
    pltpu.sync_copy(tgt_v, targets_out.at[pl.ds(tb, TPW)])

    pltpu.sync_copy(zsum_hbm.at[pl.ds(sid * RPS, RPS)],
                    acc_sh.at[pl.ds(sid * RPS, RPS)])
    z16 = jnp.zeros((16,), jnp.float32)
    for r in range(CP // 16):
        hist_v[pl.ds(r * 16, 16)] = z16
    plsc.subcore_barrier()

    ones16 = jnp.ones((16,), jnp.float32)

    def bump(idx_ref, n):
        for r in range(n // 16):
            iv = idx_ref[pl.ds(r * 16, 16)]
            plsc.addupdate_scatter(hist_v, [iv], ones16)

    rows_b = (rows0_v, rows1_v, rows2_v, rows3_v)
    idx_b = (idx0_v, idx1_v, idx2_v, idx3_v)
    ls_b = (ls0, ls1, ls2, ls3)
    ss_b = (ss0, ss1, ss2, ss3)
    NQ = 4
    NR = (ITERS + NQ - 1) // NQ

    def round4(i4, carry):
        for k in range(NQ):
            j = w + (i4 * NQ + k) * NW

            @pl.when(jnp.logical_and(i4 > 0, j < NFULL))
            def _(k=k):
                pltpu.make_async_copy(
                    rows_b[k], acc_sh.at[idx_b[k]], ss_b[k]).wait()

            @pl.when(j < NFULL)
            def _(k=k, j=j):
                pltpu.async_copy(
                    features_hbm.at[pl.ds(j * CH, CH)], rows_b[k], ls_b[k])
                pltpu.async_copy(
                    labels_hbm.at[pl.ds(j * CH, CH)], idx_b[k], ls_b[k])

        for k in range(NQ):
            j = w + (i4 * NQ + k) * NW

            @pl.when(j < NFULL)
            def _(k=k, j=j):
                pltpu.make_async_copy(
                    features_hbm.at[pl.ds(j * CH, CH)], rows_b[k],
                    ls_b[k]).wait()
                pltpu.make_async_copy(
                    labels_hbm.at[pl.ds(j * CH, CH)], idx_b[k],
                    ls_b[k]).wait()
                pltpu.async_copy(
                    rows_b[k], acc_sh.at[idx_b[k]], ss_b[k], add=True)
                bump(idx_b[k], CH)

        return carry

    lax.fori_loop(0, NR, round4, 0)
    for k in range(NQ):
        pltpu.make_async_copy(
            rows_b[k], acc_sh.at[idx_b[k]], ss_b[k]).wait()

    @pl.when(w == NW - 1)
    def _():
        pltpu.sync_copy(features_hbm.at[pl.ds(TAIL_OFF, TAIL)],
                        rows0_v.at[pl.ds(0, TAIL)])
        pltpu.sync_copy(labels_hbm.at[pl.ds(TAIL_OFF, TAIL)], idxt_v)
        pltpu.sync_copy(rows0_v.at[pl.ds(0, TAIL)], acc_sh.at[idxt_v],
                        add=True)
        bump(idxt_v, TAIL)

    plsc.subcore_barrier()

    base = cid * CP + sid * RPS
    pltpu.sync_copy(acc_sh.at[pl.ds(sid * RPS, RPS)],
                    sums_out.at[pl.ds(base, RPS)])
    pltpu.sync_copy(hist_v, cnts_out.at[w])


@functools.cache
def _get_sc_call():
  return pl.kernel(
    _sc_body,
    out_type=[
        jax.ShapeDtypeStruct((NC * CP, D), jnp.float32),
        jax.ShapeDtypeStruct((NW, CP), jnp.float32),
        jax.ShapeDtypeStruct((B,), jnp.int32),
    ],
    mesh=plsc.VectorSubcoreMesh(
        core_axis_name="c", subcore_axis_name="s",
        num_cores=NC, num_subcores=NS),
    scratch_types=[
        pltpu.VMEM((CH, D), jnp.float32),
        pltpu.VMEM((CH, D), jnp.float32),
        pltpu.VMEM((CH, D), jnp.float32),
        pltpu.VMEM((CH, D), jnp.float32),
        pltpu.VMEM((CH,), jnp.int32),
        pltpu.VMEM((CH,), jnp.int32),
        pltpu.VMEM((CH,), jnp.int32),
        pltpu.VMEM((CH,), jnp.int32),
        pltpu.VMEM((TAIL,), jnp.int32),
        pltpu.VMEM((TPW,), jnp.int32),
        pltpu.VMEM((TPW,), jnp.int32),
        pltpu.VMEM((CP,), jnp.float32),
        pltpu.VMEM_SHARED((CP, D), jnp.float32),
        pltpu.SemaphoreType.DMA,
        pltpu.SemaphoreType.DMA,
        pltpu.SemaphoreType.DMA,
        pltpu.SemaphoreType.DMA,
        pltpu.SemaphoreType.DMA,
        pltpu.SemaphoreType.DMA,
        pltpu.SemaphoreType.DMA,
        pltpu.SemaphoreType.DMA,
        pltpu.SemaphoreType.DMA,
    ],
    compiler_params=pltpu.CompilerParams(needs_layout_passes=False),
  )


def _tc_body(feat_ref, sums_ref, cnts_ref, tgt_ref, out_ref):
    feat = feat_ref[...]
    nrm = jnp.sqrt(jnp.sum(feat * feat, axis=1, keepdims=True))
    x = feat / jnp.maximum(nrm, 1e-12)

    cs = sums_ref[0:CP, :] + sums_ref[CP:2 * CP, :]

    sim = lax.dot_general(
        x, cs, (((1,), (1,)), ((), ())),
        preferred_element_type=jnp.float32,
        precision=lax.Precision.HIGHEST) * (1.0 / TEMP)

    onesk = jnp.ones((1, NW), jnp.float32)
    cnt_row = lax.dot_general(
        onesk, cnts_ref[...], (((1,), (0,)), ((), ())),
        preferred_element_type=jnp.float32,
        precision=lax.Precision.HIGHEST)

    mask = (cnt_row > 0.0).astype(jnp.float32)
    denom = mask * cnt_row + (1.0 - mask)
    simd = sim / denom
    exps = jnp.exp(simd) * mask
    ssum = jnp.sum(exps, axis=1, keepdims=True) + 1e-06
    p = exps / ssum
    logp = jnp.log(p + 1e-06)

    tgt = tgt_ref[...]
    oh = (lax.broadcasted_iota(jnp.int32, (B, CP), 1) == tgt)
    loss = -jnp.sum(jnp.where(oh, logp, 0.0)) * (1.0 / B)
    out_ref[0, 0] = loss


_tc_call = pl.pallas_call(
    _tc_body,
    out_shape=jax.ShapeDtypeStruct((1, 1), jnp.float32),
    out_specs=pl.BlockSpec(memory_space=pltpu.SMEM),
)


def kernel(feat, indexes, features, labels):
    zsum = jnp.zeros((CP, D), jnp.float32)
    sums, cnts, targets = _get_sc_call()(
        features, labels, indexes, zsum)
    loss = _tc_call(feat, sums, cnts, targets.reshape(B, 1))
    return loss.reshape(())


# speedup vs baseline: 1.6064x; 1.0261x over previous
import functools

import jax
import jax.numpy as jnp
from jax import lax
from jax.experimental import pallas as pl
from jax.experimental.pallas import tpu as pltpu
from jax.experimental.pallas import tpu_sc as plsc

B = 512
M = 100000
D = 128
C = 1000
TEMP = 0.05

NC = 2
NS = 16
NW = NC * NS

CP = 1024
RPS = CP // NS
CH = 128
NFULL = M // CH
TAIL = M - NFULL * CH
TAIL_OFF = NFULL * CH
ITERS = (NFULL + NW - 1) // NW
CW = 128
TPW = B // NW


def _sc_body(features_hbm, labels_hbm, indexes_hbm,
             zsum_hbm,
             sums_out, cnts_out, targets_out,
             rows0_v, rows1_v, rows2_v, rows3_v,
             idx0_v, idx1_v, idx2_v, idx3_v, idxt_v, tidx_v,
             tgt_v, hist_v, acc_sh, sem,
             ls0, ls1, ls2, ls3, ss0, ss1, ss2, ss3):
    cid = lax.axis_index("c")
    sid = lax.axis_index("s")
    w = cid * NS + sid

    tb = w * TPW
    pltpu.sync_copy(indexes_hbm.at[pl.ds(tb, TPW)], tidx_v)
    pltpu.async_copy(labels_hbm.at[tidx_v], tgt_v, sem).wait()
    pltpu.sync_copy(tgt_v, targets_out.at[w])

    pltpu.sync_copy(zsum_hbm.at[pl.ds(sid * RPS, RPS)],
                    acc_sh.at[pl.ds(sid * RPS, RPS)])
    z16 = jnp.zeros((16,), jnp.float32)
    for r in range(CP // 16):
        hist_v[pl.ds(r * 16, 16)] = z16
    plsc.subcore_barrier()

    ones16 = jnp.ones((16,), jnp.float32)

    def bump(idx_ref, n):
        for r in range(n // 16):
            iv = idx_ref[pl.ds(r * 16, 16)]
            plsc.addupdate_scatter(hist_v, [iv], ones16)

    rows_b = (rows0_v, rows1_v, rows2_v, rows3_v)
    idx_b = (idx0_v, idx1_v, idx2_v, idx3_v)
    ls_b = (ls0, ls1, ls2, ls3)
    ss_b = (ss0, ss1, ss2, ss3)
    NQ = 4
    NR = (ITERS + NQ - 1) // NQ

    def round4(i4, carry):
        for k in range(NQ):
            j = w + (i4 * NQ + k) * NW

            @pl.when(jnp.logical_and(i4 > 0, j < NFULL))
            def _(k=k):
                pltpu.make_async_copy(
                    rows_b[k], acc_sh.at[idx_b[k]], ss_b[k]).wait()

            @pl.when(j < NFULL)
            def _(k=k, j=j):
                pltpu.async_copy(
                    features_hbm.at[pl.ds(j * CH, CH)], rows_b[k], ls_b[k])
                pltpu.async_copy(
                    labels_hbm.at[pl.ds(j * CH, CH)], idx_b[k], ls_b[k])

        for k in range(NQ):
            j = w + (i4 * NQ + k) * NW

            @pl.when(j < NFULL)
            def _(k=k, j=j):
                pltpu.make_async_copy(
                    features_hbm.at[pl.ds(j * CH, CH)], rows_b[k],
                    ls_b[k]).wait()
                pltpu.make_async_copy(
                    labels_hbm.at[pl.ds(j * CH, CH)], idx_b[k],
                    ls_b[k]).wait()
                pltpu.async_copy(
                    rows_b[k], acc_sh.at[idx_b[k]], ss_b[k], add=True)
                bump(idx_b[k], CH)

        return carry

    lax.fori_loop(0, NR, round4, 0)
    for k in range(NQ):
        pltpu.make_async_copy(
            rows_b[k], acc_sh.at[idx_b[k]], ss_b[k]).wait()

    @pl.when(w == NW - 1)
    def _():
        pltpu.sync_copy(features_hbm.at[pl.ds(TAIL_OFF, TAIL)],
                        rows0_v.at[pl.ds(0, TAIL)])
        pltpu.sync_copy(labels_hbm.at[pl.ds(TAIL_OFF, TAIL)], idxt_v)
        pltpu.sync_copy(rows0_v.at[pl.ds(0, TAIL)], acc_sh.at[idxt_v],
                        add=True)
        bump(idxt_v, TAIL)

    plsc.subcore_barrier()

    base = cid * CP + sid * RPS
    pltpu.sync_copy(acc_sh.at[pl.ds(sid * RPS, RPS)],
                    sums_out.at[pl.ds(base, RPS)])
    pltpu.sync_copy(hist_v, cnts_out.at[w])


@functools.cache
def _get_sc_call():
  return pl.kernel(
    _sc_body,
    out_type=[
        jax.ShapeDtypeStruct((NC * CP, D), jnp.float32),
        jax.ShapeDtypeStruct((NW, CP), jnp.float32),
        jax.ShapeDtypeStruct((NW, TPW), jnp.int32),
    ],
    mesh=plsc.VectorSubcoreMesh(
        core_axis_name="c", subcore_axis_name="s",
        num_cores=NC, num_subcores=NS),
    scratch_types=[
        pltpu.VMEM((CH, D), jnp.float32),
        pltpu.VMEM((CH, D), jnp.float32),
        pltpu.VMEM((CH, D), jnp.float32),
        pltpu.VMEM((CH, D), jnp.float32),
        pltpu.VMEM((CH,), jnp.int32),
        pltpu.VMEM((CH,), jnp.int32),
        pltpu.VMEM((CH,), jnp.int32),
        pltpu.VMEM((CH,), jnp.int32),
        pltpu.VMEM((TAIL,), jnp.int32),
        pltpu.VMEM((TPW,), jnp.int32),
        pltpu.VMEM((TPW,), jnp.int32),
        pltpu.VMEM((CP,), jnp.float32),
        pltpu.VMEM_SHARED((CP, D), jnp.float32),
        pltpu.SemaphoreType.DMA,
        pltpu.SemaphoreType.DMA,
        pltpu.SemaphoreType.DMA,
        pltpu.SemaphoreType.DMA,
        pltpu.SemaphoreType.DMA,
        pltpu.SemaphoreType.DMA,
        pltpu.SemaphoreType.DMA,
        pltpu.SemaphoreType.DMA,
        pltpu.SemaphoreType.DMA,
    ],
    compiler_params=pltpu.CompilerParams(needs_layout_passes=False),
  )


def _tc_body(feat_ref, sums_ref, cnts_ref, tgt_ref, out_ref):
    feat = feat_ref[...]
    nrm = jnp.sqrt(jnp.sum(feat * feat, axis=1, keepdims=True))
    x = feat / jnp.maximum(nrm, 1e-12)

    cs = sums_ref[0:CP, :] + sums_ref[CP:2 * CP, :]

    sim = lax.dot_general(
        x, cs, (((1,), (1,)), ((), ())),
        preferred_element_type=jnp.float32,
        precision=lax.Precision.HIGHEST) * (1.0 / TEMP)

    onesk = jnp.ones((1, NW), jnp.float32)
    cnt_row = lax.dot_general(
        onesk, cnts_ref[...], (((1,), (0,)), ((), ())),
        preferred_element_type=jnp.float32,
        precision=lax.Precision.HIGHEST)

    mask = (cnt_row > 0.0).astype(jnp.float32)
    denom = mask * cnt_row + (1.0 - mask)
    simd = sim / denom
    exps = jnp.exp(simd) * mask
    ssum = jnp.sum(exps, axis=1, keepdims=True) + 1e-06
    p = exps / ssum
    logp = jnp.log(p + 1e-06)

    tgt = tgt_ref[...].reshape(NW, TPW, 1)
    logp3 = logp.reshape(NW, TPW, CP)
    oh = (lax.broadcasted_iota(jnp.int32, (NW, TPW, CP), 2) == tgt)
    loss = -jnp.sum(jnp.where(oh, logp3, 0.0)) * (1.0 / B)
    out_ref[0, 0] = loss


_tc_call = pl.pallas_call(
    _tc_body,
    out_shape=jax.ShapeDtypeStruct((1, 1), jnp.float32),
    out_specs=pl.BlockSpec(memory_space=pltpu.SMEM),
)


def kernel(feat, indexes, features, labels):
    zsum = jnp.zeros((CP, D), jnp.float32)
    sums, cnts, targets = _get_sc_call()(
        features, labels, indexes, zsum)
    loss = _tc_call(feat, sums, cnts, targets)
    return loss.reshape(())


# speedup vs baseline: 1.6365x; 1.0187x over previous
import functools

import jax
import jax.numpy as jnp
from jax import lax
from jax.experimental import pallas as pl
from jax.experimental.pallas import tpu as pltpu
from jax.experimental.pallas import tpu_sc as plsc

B = 512
M = 100000
D = 128
C = 1000
TEMP = 0.05

NC = 2
NS = 16
NW = NC * NS

CP = 1024
RPS = CP // NS
CH = 128
NFULL = M // CH
TAIL = M - NFULL * CH
TAIL_OFF = NFULL * CH
ITERS = (NFULL + NW - 1) // NW
CW = 128
TPW = B // NW


def _sc_body(features_hbm, labels_hbm, indexes_hbm,
             zsum_hbm,
             sums_out, cnts_out, targets_out,
             rows0_v, rows1_v, rows2_v, rows3_v, rows4_v, rows5_v,
             idx0_v, idx1_v, idx2_v, idx3_v, idx4_v, idx5_v,
             idxt_v, tidx_v,
             tgt_v, hist_v, acc_sh, sem,
             ls0, ls1, ls2, ls3, ls4, ls5, ss0, ss1, ss2, ss3, ss4, ss5):
    cid = lax.axis_index("c")
    sid = lax.axis_index("s")
    w = cid * NS + sid

    pltpu.sync_copy(zsum_hbm.at[pl.ds(sid * RPS, RPS)],
                    acc_sh.at[pl.ds(sid * RPS, RPS)])
    z16 = jnp.zeros((16,), jnp.float32)
    for r in range(CP // 16):
        hist_v[pl.ds(r * 16, 16)] = z16
    plsc.subcore_barrier()

    ones16 = jnp.ones((16,), jnp.float32)

    def bump(idx_ref, n):
        for r in range(n // 16):
            iv = idx_ref[pl.ds(r * 16, 16)]
            plsc.addupdate_scatter(hist_v, [iv], ones16)

    rows_b = (rows0_v, rows1_v, rows2_v, rows3_v, rows4_v, rows5_v)
    idx_b = (idx0_v, idx1_v, idx2_v, idx3_v, idx4_v, idx5_v)
    ls_b = (ls0, ls1, ls2, ls3, ls4, ls5)
    ss_b = (ss0, ss1, ss2, ss3, ss4, ss5)
    NQ = 6
    NR = (ITERS + NQ - 1) // NQ

    def round4(i4, carry):
        for k in range(NQ):
            j = w + (i4 * NQ + k) * NW

            @pl.when(jnp.logical_and(i4 > 0, j < NFULL))
            def _(k=k):
                pltpu.make_async_copy(
                    rows_b[k], acc_sh.at[idx_b[k]], ss_b[k]).wait()

            @pl.when(j < NFULL)
            def _(k=k, j=j):
                pltpu.async_copy(
                    features_hbm.at[pl.ds(j * CH, CH)], rows_b[k], ls_b[k])
                pltpu.async_copy(
                    labels_hbm.at[pl.ds(j * CH, CH)], idx_b[k], ls_b[k])

        for k in range(NQ):
            j = w + (i4 * NQ + k) * NW

            @pl.when(j < NFULL)
            def _(k=k, j=j):
                pltpu.make_async_copy(
                    features_hbm.at[pl.ds(j * CH, CH)], rows_b[k],
                    ls_b[k]).wait()
                pltpu.make_async_copy(
                    labels_hbm.at[pl.ds(j * CH, CH)], idx_b[k],
                    ls_b[k]).wait()
                pltpu.async_copy(
                    rows_b[k], acc_sh.at[idx_b[k]], ss_b[k], add=True)
                bump(idx_b[k], CH)

        return carry

    lax.fori_loop(0, NR, round4, 0)

    tb = w * TPW
    pltpu.sync_copy(indexes_hbm.at[pl.ds(tb, TPW)], tidx_v)
    pltpu.async_copy(labels_hbm.at[tidx_v], tgt_v, sem).wait()
    pltpu.sync_copy(tgt_v, targets_out.at[w])

    for k in range(NQ):
        pltpu.make_async_copy(
            rows_b[k], acc_sh.at[idx_b[k]], ss_b[k]).wait()

    @pl.when(w == NW - 1)
    def _():
        pltpu.sync_copy(features_hbm.at[pl.ds(TAIL_OFF, TAIL)],
                        rows0_v.at[pl.ds(0, TAIL)])
        pltpu.sync_copy(labels_hbm.at[pl.ds(TAIL_OFF, TAIL)], idxt_v)
        pltpu.sync_copy(rows0_v.at[pl.ds(0, TAIL)], acc_sh.at[idxt_v],
                        add=True)
        bump(idxt_v, TAIL)

    plsc.subcore_barrier()

    base = cid * CP + sid * RPS
    pltpu.sync_copy(acc_sh.at[pl.ds(sid * RPS, RPS)],
                    sums_out.at[pl.ds(base, RPS)])
    pltpu.sync_copy(hist_v, cnts_out.at[w])


@functools.cache
def _get_sc_call():
  return pl.kernel(
    _sc_body,
    out_type=[
        jax.ShapeDtypeStruct((NC * CP, D), jnp.float32),
        jax.ShapeDtypeStruct((NW, CP), jnp.float32),
        jax.ShapeDtypeStruct((NW, TPW), jnp.int32),
    ],
    mesh=plsc.VectorSubcoreMesh(
        core_axis_name="c", subcore_axis_name="s",
        num_cores=NC, num_subcores=NS),
    scratch_types=[
        pltpu.VMEM((CH, D), jnp.float32),
        pltpu.VMEM((CH, D), jnp.float32),
        pltpu.VMEM((CH, D), jnp.float32),
        pltpu.VMEM((CH, D), jnp.float32),
        pltpu.VMEM((CH, D), jnp.float32),
        pltpu.VMEM((CH, D), jnp.float32),
        pltpu.VMEM((CH,), jnp.int32),
        pltpu.VMEM((CH,), jnp.int32),
        pltpu.VMEM((CH,), jnp.int32),
        pltpu.VMEM((CH,), jnp.int32),
        pltpu.VMEM((CH,), jnp.int32),
        pltpu.VMEM((CH,), jnp.int32),
        pltpu.VMEM((TAIL,), jnp.int32),
        pltpu.VMEM((TPW,), jnp.int32),
        pltpu.VMEM((TPW,), jnp.int32),
        pltpu.VMEM((CP,), jnp.float32),
        pltpu.VMEM_SHARED((CP, D), jnp.float32),
        pltpu.SemaphoreType.DMA,
        pltpu.SemaphoreType.DMA,
        pltpu.SemaphoreType.DMA,
        pltpu.SemaphoreType.DMA,
        pltpu.SemaphoreType.DMA,
        pltpu.SemaphoreType.DMA,
        pltpu.SemaphoreType.DMA,
        pltpu.SemaphoreType.DMA,
        pltpu.SemaphoreType.DMA,
        pltpu.SemaphoreType.DMA,
        pltpu.SemaphoreType.DMA,
        pltpu.SemaphoreType.DMA,
        pltpu.SemaphoreType.DMA,
    ],
    compiler_params=pltpu.CompilerParams(needs_layout_passes=False),
  )


def _tc_body(feat_ref, sums_ref, cnts_ref, tgt_ref, out_ref):
    feat = feat_ref[...]
    nrm = jnp.sqrt(jnp.sum(feat * feat, axis=1, keepdims=True))
    x = feat / jnp.maximum(nrm, 1e-12)

    cs = sums_ref[0:CP, :] + sums_ref[CP:2 * CP, :]

    sim = lax.dot_general(
        x, cs, (((1,), (1,)), ((), ())),
        preferred_element_type=jnp.float32,
        precision=lax.Precision.HIGHEST) * (1.0 / TEMP)

    onesk = jnp.ones((1, NW), jnp.float32)
    cnt_row = lax.dot_general(
        onesk, cnts_ref[...], (((1,), (0,)), ((), ())),
        preferred_element_type=jnp.float32,
        precision=lax.Precision.HIGHEST)

    mask = (cnt_row > 0.0).astype(jnp.float32)
    denom = mask * cnt_row + (1.0 - mask)
    simd = sim / denom
    exps = jnp.exp(simd) * mask
    ssum = jnp.sum(exps, axis=1, keepdims=True) + 1e-06
    p = exps / ssum
    logp = jnp.log(p + 1e-06)

    tgt = tgt_ref[...].reshape(NW, TPW, 1)
    logp3 = logp.reshape(NW, TPW, CP)
    oh = (lax.broadcasted_iota(jnp.int32, (NW, TPW, CP), 2) == tgt)
    loss = -jnp.sum(jnp.where(oh, logp3, 0.0)) * (1.0 / B)
    out_ref[0, 0] = loss


_tc_call = pl.pallas_call(
    _tc_body,
    out_shape=jax.ShapeDtypeStruct((1, 1), jnp.float32),
    out_specs=pl.BlockSpec(memory_space=pltpu.SMEM),
)


def kernel(feat, indexes, features, labels):
    zsum = jnp.zeros((CP, D), jnp.float32)
    sums, cnts, targets = _get_sc_call()(
        features, labels, indexes, zsum)
    loss = _tc_call(feat, sums, cnts, targets)
    return loss.reshape(())
